# baseline (speedup 1.0000x reference)
import functools

import jax
import jax.numpy as jnp
from jax import lax
from jax.experimental import pallas as pl
from jax.experimental.pallas import tpu as pltpu
from jax.experimental.pallas import tpu_sc as plsc

N = 10000
NSUB = 2500
E = 320000
C = 128

NC, NS, L = 2, 16, 16
NT = NC * NS
RNG = 313
DUMP = RNG
FCH = 2000
NFCH = E // FCH
GE = 256
CAP = E + 8192

_MESH = plsc.VectorSubcoreMesh(core_axis_name="c", subcore_axis_name="s",
                               num_cores=NC, num_subcores=NS)
_SC_PARAMS = pltpu.CompilerParams(needs_layout_passes=False,
                                  use_tc_tiling_on_sc=False)

GW = 25
GROWS = N // GW


@functools.partial(
    pl.kernel,
    out_type=jax.ShapeDtypeStruct((N, 2 * C), jnp.float32),
    mesh=_MESH,
    compiler_params=_SC_PARAMS,
    scratch_types=[
        pltpu.VMEM((GROWS,), jnp.int32),
        pltpu.VMEM((GROWS, 2 * C), jnp.float32),
        pltpu.SemaphoreType.DMA,
    ],
)
def _up_gather(tab_hbm, idx_hbm, out_hbm, idx_v, rows_v, sem):
    wid = lax.axis_index("c") * NS + lax.axis_index("s")

    @pl.when(wid < GW)
    def _():
        base = wid * GROWS
        pltpu.sync_copy(idx_hbm.at[pl.ds(base, GROWS)], idx_v)
        pltpu.async_copy(tab_hbm.at[idx_v], rows_v, sem).wait()
        pltpu.sync_copy(rows_v, out_hbm.at[pl.ds(base, GROWS)])


@functools.partial(
    pl.kernel,
    out_type=(jax.ShapeDtypeStruct((NT, CAP), jnp.int32),
              jax.ShapeDtypeStruct((NT, L), jnp.int32)),
    mesh=_MESH,
    compiler_params=_SC_PARAMS,
    scratch_types=[
        pltpu.VMEM((FCH,), jnp.int32),
        pltpu.VMEM((FCH,), jnp.int32),
        pltpu.VMEM((FCH + 2 * L,), jnp.int32),
        pltpu.VMEM((L,), jnp.int32),
    ],
)
def _filter(src_hbm, dst_hbm, lists_hbm, cnt_hbm, src_v, dst_v, stage, cnt_v):
    t = lax.axis_index("c") * NS + lax.axis_index("s")
    base = t * RNG
    dump = jnp.full((L,), DUMP, jnp.int32)

    def chunk_body(ci, goff):
        pltpu.sync_copy(src_hbm.at[pl.ds(ci * FCH, FCH)], src_v)
        pltpu.sync_copy(dst_hbm.at[pl.ds(ci * FCH, FCH)], dst_v)

        def scan_body(v, off):
            dv = dst_v[pl.ds(v * L, L)]
            sv = src_v[pl.ds(v * L, L)]
            dloc = dv - base
            m = (dloc >= 0) & (dloc < RNG)
            packed = (sv << 9) | dloc
            plsc.store_compressed(stage.at[pl.ds(off, L)], packed, mask=m)
            return off + jnp.sum(m.astype(jnp.int32))

        off = lax.fori_loop(0, FCH // L, scan_body, 0)
        stage[pl.ds(off, L)] = dump
        off16 = ((off + L - 1) // L) * L
        pltpu.sync_copy(stage.at[pl.ds(0, FCH + L)],
                        lists_hbm.at[t].at[pl.ds(pl.multiple_of(goff, L),
                                                 FCH + L)])
        return goff + off16

    goff = lax.fori_loop(0, NFCH, chunk_body, 0)

    def dump_body(v, _):
        stage[pl.ds(v * L, L)] = dump
        return 0

    lax.fori_loop(0, GE // L, dump_body, 0)
    pltpu.sync_copy(stage.at[pl.ds(0, GE)],
                    lists_hbm.at[t].at[pl.ds(pl.multiple_of(goff, L), GE)])
    cntg = ((goff + GE - 1) // GE) * GE
    cnt_v[...] = jnp.zeros((L,), jnp.int32) + cntg
    pltpu.sync_copy(cnt_v, cnt_hbm.at[t])


NK = C // L


@functools.partial(
    pl.kernel,
    out_type=jax.ShapeDtypeStruct((NT, NK, RNG * L), jnp.float32),
    mesh=_MESH,
    compiler_params=_SC_PARAMS,
    scratch_types=[pltpu.VMEM(((RNG + 1) * L,), jnp.float32)] * NK + [
        pltpu.VMEM((2, GE), jnp.int32),
        pltpu.VMEM((2, GE), jnp.int32),
        pltpu.VMEM((2, GE, C), jnp.float32),
        pltpu.VMEM((L,), jnp.int32),
        pltpu.SemaphoreType.DMA((2,)),
    ],
)
def _seg_max(a_hbm, lists_hbm, cnt_hbm, out_hbm, *refs):
    accs = refs[:NK]
    pk_v, idx_v, row_v, cnt_v, sem = refs[NK:]
    t = lax.axis_index("c") * NS + lax.axis_index("s")

    iota = lax.iota(jnp.int32, L)
    neg_inf = jnp.full((L,), -jnp.inf, jnp.float32)
    zero = jnp.zeros((L,), jnp.int32)

    def init(k, _):
        for a in accs:
            a[pl.ds(k * L, L)] = neg_inf
        return 0

    lax.fori_loop(0, (RNG + 1) * L // L, init, 0)

    pltpu.sync_copy(cnt_hbm.at[t], cnt_v)
    ng = jnp.max(cnt_v[...]) // GE

    def fetch(g, b):
        pltpu.sync_copy(lists_hbm.at[t].at[pl.ds(g * GE, GE)], pk_v.at[b])

        def unpack(v, _):
            idx_v[b, pl.ds(v * L, L)] = pk_v[b, pl.ds(v * L, L)] >> 9
            return 0

        lax.fori_loop(0, GE // L, unpack, 0)
        pltpu.async_copy(a_hbm.at[idx_v.at[b]], row_v.at[b], sem.at[b])

    @pl.when(ng > 0)
    def _():
        fetch(0, 0)

        def group_body(g, _):
            b = g & 1

            @pl.when(g + 1 < ng)
            def _():
                fetch(g + 1, 1 - b)

            pltpu.make_async_copy(a_hbm.at[idx_v.at[b]], row_v.at[b],
                                  sem.at[b]).wait()

            def blk_body(jj, _):
                pkvec = pk_v[b, pl.ds(jj * L, L)] & 511
                for l in range(L):
                    abase = pkvec[l] * L
                    j = jj * L + l
                    vs = [row_v[b, j, pl.ds(k * L, L)] for k in range(NK)]
                    olds = [accs[k][pl.ds(abase, L)] for k in range(NK)]
                    for k in range(NK):
                        accs[k][pl.ds(abase, L)] = jnp.maximum(olds[k], vs[k])
                return 0

            lax.fori_loop(0, GE // L, blk_body, 0)
            return 0

        lax.fori_loop(0, ng, group_body, 0)

    for k in range(NK):
        pltpu.sync_copy(accs[k].at[pl.ds(0, RNG * L)], out_hbm.at[t, k])


def _segment_max(a, lists, cnts):
    part = _seg_max(a, lists, cnts)
    part = part.reshape(NT, NK, RNG, L).transpose(0, 2, 1, 3)
    return part.reshape(NT * RNG, C)[:N]


def kernel(sub_x, x, edge_index, up_idx, W_mix, b_mix, W_res, b_res,
           gamma_res, beta_res):
    src = edge_index[0]
    dst = edge_index[1]
    lists, cnts = _filter(src, dst)

    W1m, W2m = W_mix[:2 * C], W_mix[2 * C:]
    Dm = W1m - W2m
    sub_proj = jnp.concatenate([sub_x @ W2m[C:], sub_x @ Dm[C:]], axis=1)
    up = _up_gather(sub_proj, up_idx)
    A1 = x @ W2m[:C] + up[:, :C]
    B1 = x @ Dm[:C] + up[:, C:2 * C] + b_mix

    s1 = _segment_max(A1, lists, cnts) + B1
    h = jnp.where(jnp.isfinite(s1), s1, 0.0)

    W1r, W2r = W_res[:C], W_res[C:]
    A2 = h @ W2r
    B2 = h @ (W1r - W2r) + b_res
    s2 = _segment_max(A2, lists, cnts) + B2
    h_res = jnp.where(jnp.isfinite(s2), s2, 0.0)

    mu = jnp.mean(h_res, axis=0, keepdims=True)
    var = jnp.var(h_res, axis=0, keepdims=True)
    hn = (h_res - mu) / jnp.sqrt(var + 1e-5) * gamma_res + beta_res
    return jax.nn.relu(hn) + h


# speedup vs baseline: 1.0409x; 1.0409x over previous
import functools

import jax
import jax.numpy as jnp
from jax import lax
from jax.experimental import pallas as pl
from jax.experimental.pallas import tpu as pltpu
from jax.experimental.pallas import tpu_sc as plsc

N = 10000
NSUB = 2500
E = 320000
C = 128

NC, NS, L = 2, 16, 16
NT = NC * NS
RNG = 313
DUMP = RNG
FCH = 2000
NFCH = E // FCH
GE = 128
CAP = E + 8192

_MESH = plsc.VectorSubcoreMesh(core_axis_name="c", subcore_axis_name="s",
                               num_cores=NC, num_subcores=NS)
_SC_PARAMS = pltpu.CompilerParams(needs_layout_passes=False,
                                  use_tc_tiling_on_sc=False)

GW = 25
GROWS = N // GW


@functools.partial(
    pl.kernel,
    out_type=jax.ShapeDtypeStruct((N, 2 * C), jnp.float32),
    mesh=_MESH,
    compiler_params=_SC_PARAMS,
    scratch_types=[
        pltpu.VMEM((GROWS,), jnp.int32),
        pltpu.VMEM((GROWS, 2 * C), jnp.float32),
        pltpu.SemaphoreType.DMA,
    ],
)
def _up_gather(tab_hbm, idx_hbm, out_hbm, idx_v, rows_v, sem):
    wid = lax.axis_index("c") * NS + lax.axis_index("s")

    @pl.when(wid < GW)
    def _():
        base = wid * GROWS
        pltpu.sync_copy(idx_hbm.at[pl.ds(base, GROWS)], idx_v)
        pltpu.async_copy(tab_hbm.at[idx_v], rows_v, sem).wait()
        pltpu.sync_copy(rows_v, out_hbm.at[pl.ds(base, GROWS)])


@functools.partial(
    pl.kernel,
    out_type=(jax.ShapeDtypeStruct((NT, CAP), jnp.int32),
              jax.ShapeDtypeStruct((NT, L), jnp.int32)),
    mesh=_MESH,
    compiler_params=_SC_PARAMS,
    scratch_types=[
        pltpu.VMEM((FCH,), jnp.int32),
        pltpu.VMEM((FCH,), jnp.int32),
        pltpu.VMEM((FCH + 2 * L,), jnp.int32),
        pltpu.VMEM((L,), jnp.int32),
    ],
)
def _filter(src_hbm, dst_hbm, lists_hbm, cnt_hbm, src_v, dst_v, stage, cnt_v):
    t = lax.axis_index("c") * NS + lax.axis_index("s")
    base = t * RNG
    dump = jnp.full((L,), DUMP, jnp.int32)

    def chunk_body(ci, goff):
        pltpu.sync_copy(src_hbm.at[pl.ds(ci * FCH, FCH)], src_v)
        pltpu.sync_copy(dst_hbm.at[pl.ds(ci * FCH, FCH)], dst_v)

        def scan_body(v, off):
            dv = dst_v[pl.ds(v * L, L)]
            sv = src_v[pl.ds(v * L, L)]
            dloc = dv - base
            m = (dloc >= 0) & (dloc < RNG)
            packed = (sv << 9) | dloc
            plsc.store_compressed(stage.at[pl.ds(off, L)], packed, mask=m)
            return off + jnp.sum(m.astype(jnp.int32))

        off = lax.fori_loop(0, FCH // L, scan_body, 0)
        stage[pl.ds(off, L)] = dump
        off16 = ((off + L - 1) // L) * L
        pltpu.sync_copy(stage.at[pl.ds(0, FCH + L)],
                        lists_hbm.at[t].at[pl.ds(pl.multiple_of(goff, L),
                                                 FCH + L)])
        return goff + off16

    goff = lax.fori_loop(0, NFCH, chunk_body, 0)

    def dump_body(v, _):
        stage[pl.ds(v * L, L)] = dump
        return 0

    lax.fori_loop(0, GE // L, dump_body, 0)
    pltpu.sync_copy(stage.at[pl.ds(0, GE)],
                    lists_hbm.at[t].at[pl.ds(pl.multiple_of(goff, L), GE)])
    cntg = ((goff + GE - 1) // GE) * GE
    cnt_v[...] = jnp.zeros((L,), jnp.int32) + cntg
    pltpu.sync_copy(cnt_v, cnt_hbm.at[t])


NK = C // L


@functools.partial(
    pl.kernel,
    out_type=jax.ShapeDtypeStruct((NT, NK, RNG * L), jnp.float32),
    mesh=_MESH,
    compiler_params=_SC_PARAMS,
    scratch_types=[pltpu.VMEM(((RNG + 1) * L,), jnp.float32)] * NK + [
        pltpu.VMEM((2, GE), jnp.int32),
        pltpu.VMEM((2, GE), jnp.int32),
        pltpu.VMEM((2, GE, C), jnp.float32),
        pltpu.VMEM((L,), jnp.int32),
        pltpu.SemaphoreType.DMA((2,)),
    ],
)
def _seg_max(a_hbm, lists_hbm, cnt_hbm, out_hbm, *refs):
    accs = refs[:NK]
    pk_v, idx_v, row_v, cnt_v, sem = refs[NK:]
    t = lax.axis_index("c") * NS + lax.axis_index("s")

    iota = lax.iota(jnp.int32, L)
    neg_inf = jnp.full((L,), -jnp.inf, jnp.float32)
    zero = jnp.zeros((L,), jnp.int32)

    def init(k, _):
        for a in accs:
            a[pl.ds(k * L, L)] = neg_inf
        return 0

    lax.fori_loop(0, (RNG + 1) * L // L, init, 0)

    pltpu.sync_copy(cnt_hbm.at[t], cnt_v)
    ng = jnp.max(cnt_v[...]) // GE

    def fetch(g, b):
        pltpu.sync_copy(lists_hbm.at[t].at[pl.ds(g * GE, GE)], pk_v.at[b])

        def unpack(v, _):
            idx_v[b, pl.ds(v * L, L)] = pk_v[b, pl.ds(v * L, L)] >> 9
            return 0

        lax.fori_loop(0, GE // L, unpack, 0)
        pltpu.async_copy(a_hbm.at[idx_v.at[b]], row_v.at[b], sem.at[b])

    @pl.when(ng > 0)
    def _():
        fetch(0, 0)

        def group_body(g, _):
            b = g & 1

            @pl.when(g + 1 < ng)
            def _():
                fetch(g + 1, 1 - b)

            pltpu.make_async_copy(a_hbm.at[idx_v.at[b]], row_v.at[b],
                                  sem.at[b]).wait()

            def blk_body(jj, _):
                pkvec = pk_v[b, pl.ds(jj * L, L)] & 511
                for l in range(L):
                    abase = pkvec[l] * L
                    j = jj * L + l
                    vs = [row_v[b, j, pl.ds(k * L, L)] for k in range(NK)]
                    olds = [accs[k][pl.ds(abase, L)] for k in range(NK)]
                    for k in range(NK):
                        accs[k][pl.ds(abase, L)] = jnp.maximum(olds[k], vs[k])
                return 0

            lax.fori_loop(0, GE // L, blk_body, 0)
            return 0

        lax.fori_loop(0, ng, group_body, 0)

    for k in range(NK):
        pltpu.sync_copy(accs[k].at[pl.ds(0, RNG * L)], out_hbm.at[t, k])


def _segment_max(a, lists, cnts):
    part = _seg_max(a, lists, cnts)
    part = part.reshape(NT, NK, RNG, L).transpose(0, 2, 1, 3)
    return part.reshape(NT * RNG, C)[:N]


def kernel(sub_x, x, edge_index, up_idx, W_mix, b_mix, W_res, b_res,
           gamma_res, beta_res):
    src = edge_index[0]
    dst = edge_index[1]
    lists, cnts = _filter(src, dst)

    W1m, W2m = W_mix[:2 * C], W_mix[2 * C:]
    Dm = W1m - W2m
    sub_proj = jnp.concatenate([sub_x @ W2m[C:], sub_x @ Dm[C:]], axis=1)
    up = _up_gather(sub_proj, up_idx)
    A1 = x @ W2m[:C] + up[:, :C]
    B1 = x @ Dm[:C] + up[:, C:2 * C] + b_mix

    s1 = _segment_max(A1, lists, cnts) + B1
    h = jnp.where(jnp.isfinite(s1), s1, 0.0)

    W1r, W2r = W_res[:C], W_res[C:]
    A2 = h @ W2r
    B2 = h @ (W1r - W2r) + b_res
    s2 = _segment_max(A2, lists, cnts) + B2
    h_res = jnp.where(jnp.isfinite(s2), s2, 0.0)

    mu = jnp.mean(h_res, axis=0, keepdims=True)
    var = jnp.var(h_res, axis=0, keepdims=True)
    hn = (h_res - mu) / jnp.sqrt(var + 1e-5) * gamma_res + beta_res
    return jax.nn.relu(hn) + h


# speedup vs baseline: 1.3762x; 1.3221x over previous
import functools

import jax
import jax.numpy as jnp
from jax import lax
from jax.experimental import pallas as pl
from jax.experimental.pallas import tpu as pltpu
from jax.experimental.pallas import tpu_sc as plsc

N = 10000
NSUB = 2500
E = 320000
C = 128

NC, NS, L = 2, 16, 16
CG = 8
NCG = C // CG
EH = E // 2
CHUNK = 2000
NCHUNK = EH // CHUNK
UNROLL = 4

_MESH = plsc.VectorSubcoreMesh(core_axis_name="c", subcore_axis_name="s",
                               num_cores=NC, num_subcores=NS)

GW = 25
GROWS = N // GW


@functools.partial(
    pl.kernel,
    out_type=jax.ShapeDtypeStruct((N, 2 * C), jnp.float32),
    mesh=_MESH,
    scratch_types=[
        pltpu.VMEM((GROWS,), jnp.int32),
        pltpu.VMEM((GROWS, 2 * C), jnp.float32),
        pltpu.SemaphoreType.DMA,
    ],
)
def _up_gather(tab_hbm, idx_hbm, out_hbm, idx_v, rows_v, sem):
    wid = lax.axis_index("c") * NS + lax.axis_index("s")

    @pl.when(wid < GW)
    def _():
        base = wid * GROWS
        pltpu.sync_copy(idx_hbm.at[pl.ds(base, GROWS)], idx_v)
        pltpu.async_copy(tab_hbm.at[idx_v], rows_v, sem).wait()
        pltpu.sync_copy(rows_v, out_hbm.at[pl.ds(base, GROWS)])


@functools.partial(
    pl.kernel,
    out_type=jax.ShapeDtypeStruct((2, NCG, N * CG), jnp.float32),
    mesh=_MESH,
    compiler_params=pltpu.CompilerParams(needs_layout_passes=False,
                                         use_tc_tiling_on_sc=False),
    scratch_types=[
        pltpu.VMEM((N * CG,), jnp.float32),
        pltpu.VMEM((CHUNK,), jnp.int32),
        pltpu.VMEM((CHUNK,), jnp.int32),
        pltpu.VMEM((CHUNK, CG), jnp.float32),
        pltpu.SemaphoreType.DMA,
    ],
)
def _seg_max(ag_hbm, src_hbm, dst_hbm, out_hbm, acc, src_v, dst_v, row_v, sem):
    half = lax.axis_index("c")
    cg = lax.axis_index("s")

    iota = lax.iota(jnp.int32, L)
    lane8 = iota & 7
    hi = (iota >> 3) & 1
    lo = 1 - hi
    neg_inf = jnp.full((L,), -jnp.inf, jnp.float32)

    def init(k, _):
        acc[pl.ds(k * L, L)] = neg_inf
        return 0

    lax.fori_loop(0, N * CG // L, init, 0)

    def chunk_body(ci, _):
        base = half * EH + ci * CHUNK
        pltpu.sync_copy(src_hbm.at[pl.ds(base, CHUNK)], src_v)
        pltpu.sync_copy(dst_hbm.at[pl.ds(base, CHUNK)], dst_v)
        pltpu.async_copy(ag_hbm.at[cg].at[src_v], row_v, sem).wait()

        def edge_body(i, _):
            for j in range(UNROLL):
                i0 = 2 * (UNROLL * i + j)
                rows = i0 + hi
                rows_sw = i0 + lo
                d = plsc.load_gather(dst_v, [rows])
                d_sw = plsc.load_gather(dst_v, [rows_sw])
                v = plsc.load_gather(row_v, [rows, lane8])
                v_sw = plsc.load_gather(row_v, [rows_sw, lane8])
                v = jnp.where(d == d_sw, jnp.maximum(v, v_sw), v)
                aidx = d * CG + lane8
                old = plsc.load_gather(acc, [aidx])
                plsc.store_scatter(acc, [aidx], jnp.maximum(old, v))
            return 0

        lax.fori_loop(0, CHUNK // (2 * UNROLL), edge_body, 0)
        return 0

    lax.fori_loop(0, NCHUNK, chunk_body, 0)
    pltpu.sync_copy(acc, out_hbm.at[half, cg])


def _segment_max(a, src, dst):
    ag = a.reshape(N, NCG, CG).transpose(1, 0, 2)
    part = _seg_max(ag, src, dst)
    agg = jnp.max(part, axis=0).reshape(NCG, N, CG)
    return agg.transpose(1, 0, 2).reshape(N, C)


def kernel(sub_x, x, edge_index, up_idx, W_mix, b_mix, W_res, b_res,
           gamma_res, beta_res):
    src = edge_index[0]
    dst = edge_index[1]

    W1m, W2m = W_mix[:2 * C], W_mix[2 * C:]
    Dm = W1m - W2m
    sub_proj = jnp.concatenate([sub_x @ W2m[C:], sub_x @ Dm[C:]], axis=1)
    up = _up_gather(sub_proj, up_idx)
    A1 = x @ W2m[:C] + up[:, :C]
    B1 = x @ Dm[:C] + up[:, C:2 * C] + b_mix

    s1 = _segment_max(A1, src, dst) + B1
    h = jnp.where(jnp.isfinite(s1), s1, 0.0)

    W1r, W2r = W_res[:C], W_res[C:]
    A2 = h @ W2r
    B2 = h @ (W1r - W2r) + b_res
    s2 = _segment_max(A2, src, dst) + B2
    h_res = jnp.where(jnp.isfinite(s2), s2, 0.0)

    mu = jnp.mean(h_res, axis=0, keepdims=True)
    var = jnp.var(h_res, axis=0, keepdims=True)
    hn = (h_res - mu) / jnp.sqrt(var + 1e-5) * gamma_res + beta_res
    return jax.nn.relu(hn) + h
